# baseline (speedup 1.0000x reference)
import functools

import jax
import jax.numpy as jnp
from jax import lax
from jax.experimental import pallas as pl
from jax.experimental.pallas import tpu as pltpu
from jax.experimental.pallas import tpu_sc as plsc

D = 128
N_GROUPS = 40000
G_PAD = 40960
NW = 32
CH = 64
T_PAD = 163840
EPW = T_PAD // NW
NCH = EPW // CH
NV = D // 16


def _proj_body(stu, conc, item, ws, wi, bs, bi, hs_o, hi_o, hc_o):
    hs_o[...] = jnp.dot(stu[...], ws[...], preferred_element_type=jnp.float32)
    hi_o[...] = jnp.dot(item[...], wi[...], preferred_element_type=jnp.float32)
    hc_o[:, :D] = (jnp.dot(conc[...], ws[...],
                           preferred_element_type=jnp.float32) + bs[...])
    hc_o[:, D:] = (jnp.dot(conc[...], wi[...],
                           preferred_element_type=jnp.float32) + bi[...])


def _project(stu_x, conc_x, item_x, ws, wi, bs, bi):
    n_stu, n_conc, n_item = stu_x.shape[0], conc_x.shape[0], item_x.shape[0]
    return pl.pallas_call(
        _proj_body,
        out_shape=[
            jax.ShapeDtypeStruct((n_stu, D), jnp.float32),
            jax.ShapeDtypeStruct((n_item, D), jnp.float32),
            jax.ShapeDtypeStruct((n_conc, 2 * D), jnp.float32),
        ],
    )(stu_x, conc_x, item_x, ws, wi, bs.reshape(1, D), bi.reshape(1, D))


_MESH = plsc.VectorSubcoreMesh(core_axis_name="c", subcore_axis_name="s")

_GDN = lax.GatherDimensionNumbers(offset_dims=(), collapsed_slice_dims=(0,),
                                  start_index_map=(0,))


def _shuffle(x, idx):
    return lax.gather(x, idx[:, None], dimension_numbers=_GDN,
                      slice_sizes=(1,),
                      mode=lax.GatherScatterMode.PROMISE_IN_BOUNDS)


@functools.partial(
    pl.kernel,
    out_type=[
        jax.ShapeDtypeStruct((2, G_PAD), jnp.float32),
        jax.ShapeDtypeStruct((2, G_PAD), jnp.float32),
    ],
    mesh=_MESH,
    scratch_types=[
        pltpu.VMEM((NCH, CH), jnp.int32),
        pltpu.VMEM((NCH, CH), jnp.int32),
        pltpu.VMEM((NCH, CH), jnp.int32),
        pltpu.VMEM((NCH, CH), jnp.int32),
        pltpu.VMEM((CH, D), jnp.float32),
        pltpu.VMEM((CH, D), jnp.float32),
        pltpu.VMEM((CH, D), jnp.float32),
        pltpu.VMEM((CH, D), jnp.float32),
        pltpu.VMEM((CH, 2 * D), jnp.float32),
        pltpu.VMEM((CH, 2 * D), jnp.float32),
        pltpu.VMEM((CH,), jnp.float32),
        pltpu.VMEM((CH,), jnp.float32),
        pltpu.VMEM((CH,), jnp.float32),
        pltpu.VMEM((D,), jnp.float32),
        pltpu.VMEM_SHARED((G_PAD,), jnp.float32),
        pltpu.VMEM_SHARED((G_PAD,), jnp.float32),
        pltpu.SemaphoreType.DMA,
        pltpu.SemaphoreType.DMA,
        pltpu.SemaphoreType.DMA,
        pltpu.SemaphoreType.DMA,
    ],
)
def _sc_edges(hs, hi, hc, wpred_hbm, zsum_hbm, zcnt_hbm,
              stu_idx, item_idx, conc_idx, mean_idx,
              sums_out, counts_out,
              stu_iv, item_iv, conc_iv, mean_iv,
              rs0, rs1, ri0, ri1, rc0, rc1, sv0, sv1,
              ones_v, w_v,
              sh_sums, sh_counts, sem0, sem1, ssem0, ssem1):
    cid = lax.axis_index("c")
    sid = lax.axis_index("s")
    wid = sid * 2 + cid
    rs, ri, rc, sv = (rs0, rs1), (ri0, ri1), (rc0, rc1), (sv0, sv1)
    sems = (sem0, sem1)
    ssems = (ssem0, ssem1)

    @pl.when(sid == 0)
    def _zero():
        pltpu.sync_copy(zsum_hbm, sh_sums)
        pltpu.sync_copy(zcnt_hbm, sh_counts)

    pltpu.sync_copy(stu_idx.at[wid], stu_iv)
    pltpu.sync_copy(item_idx.at[wid], item_iv)
    pltpu.sync_copy(conc_idx.at[wid], conc_iv)
    pltpu.sync_copy(mean_idx.at[wid], mean_iv)
    pltpu.sync_copy(wpred_hbm, w_v)
    one = jnp.full((16,), 1.0, jnp.float32)
    for j in range(CH // 16):
        ones_v[pl.ds(j * 16, 16)] = one
    wvecs = [w_v[pl.ds(j * 16, 16)] for j in range(NV)]
    lane = jnp.arange(16, dtype=jnp.int32)
    perms = [lane ^ (1 << p) for p in range(4)]

    plsc.subcore_barrier()

    def start(k, b):
        pltpu.async_copy(hs.at[stu_iv.at[k]], rs[b], sems[b])
        pltpu.async_copy(hi.at[item_iv.at[k]], ri[b], sems[b])
        pltpu.async_copy(hc.at[conc_iv.at[k]], rc[b], sems[b])

    def drain(k, b):
        pltpu.make_async_copy(hs.at[stu_iv.at[k]], rs[b], sems[b]).wait()
        pltpu.make_async_copy(hi.at[item_iv.at[k]], ri[b], sems[b]).wait()
        pltpu.make_async_copy(hc.at[conc_iv.at[k]], rc[b], sems[b]).wait()

    def drain_scatter(k, b):
        pltpu.make_async_copy(
            sv[b], sh_sums.at[mean_iv.at[k]], ssems[b]).wait()
        pltpu.make_async_copy(
            ones_v, sh_counts.at[mean_iv.at[k]], ssems[b]).wait()

    start(0, 0)

    def outer_body(k0, carry):
        for b in range(2):
            k = k0 * 2 + b
            drain(k, b)

            @pl.when(k + 1 < NCH)
            def _prefetch():
                start(k + 1, 1 - b)

            @pl.when(k >= 2)
            def _sdrain():
                drain_scatter(k - 2, b)

            mrs, mri, mrc = rs[b], ri[b], rc[b]
            msv = sv[b]

            def group_body(g, c2):
                def edge_body(e, svec):
                    t2 = g * 16 + e
                    acc = jnp.zeros((16,), jnp.float32)
                    for j in range(NV):
                        a = (mrc[t2, pl.ds(j * 16, 16)]
                             + mrs[t2, pl.ds(j * 16, 16)])
                        bb = (mrc[t2, pl.ds(D + j * 16, 16)]
                              + mri[t2, pl.ds(j * 16, 16)])
                        ea = jnp.exp(a)
                        eb = jnp.exp(bb)
                        acc = acc + wvecs[j] * ((ea - eb)
                                                / ((1.0 + ea) * (1.0 + eb)))
                    for p in perms:
                        acc = acc + _shuffle(acc, p)
                    return jnp.where(lane == e, acc, svec)

                svec = plsc.parallel_loop(
                    0, 16, unroll=4,
                    carry=jnp.zeros((16,), jnp.float32))(edge_body)
                msv[pl.ds(g * 16, 16)] = svec
                return c2

            lax.fori_loop(0, CH // 16, group_body, 0)
            pltpu.async_copy(msv, sh_sums.at[mean_iv.at[k]], ssems[b],
                             add=True)
            pltpu.async_copy(ones_v, sh_counts.at[mean_iv.at[k]], ssems[b],
                             add=True)
        return carry

    lax.fori_loop(0, NCH // 2, outer_body, 0)
    drain_scatter(NCH - 2, 0)
    drain_scatter(NCH - 1, 1)

    plsc.subcore_barrier()

    @pl.when(sid == 0)
    def _flush():
        pltpu.sync_copy(sh_sums, sums_out.at[cid])
        pltpu.sync_copy(sh_counts, counts_out.at[cid])


def _combine_body(s_ref, c_ref, b_ref, o_ref):
    tot = s_ref[0] + s_ref[1]
    cnt = jnp.maximum(c_ref[0] + c_ref[1], 1.0)
    o_ref[...] = jax.nn.sigmoid(tot / cnt + b_ref[0, 0])


def _combine(sums, counts, b_pred):
    return pl.pallas_call(
        _combine_body,
        out_shape=jax.ShapeDtypeStruct((G_PAD // D, D), jnp.float32),
        in_specs=[
            pl.BlockSpec(memory_space=pltpu.VMEM),
            pl.BlockSpec(memory_space=pltpu.VMEM),
            pl.BlockSpec(memory_space=pltpu.SMEM),
        ],
    )(sums.reshape(2, G_PAD // D, D), counts.reshape(2, G_PAD // D, D),
      b_pred.reshape(1, 1))


def kernel(stu_x, conc_x, item_x, stu_track, item_index, conc_index,
           mean_index, W_feat_stu, b_feat_stu, W_feat_item, b_feat_item,
           W_pred, b_pred):
    hs, hi, hc = _project(stu_x, conc_x, item_x,
                          W_feat_stu, W_feat_item, b_feat_stu, b_feat_item)
    t = stu_track.shape[0]
    pad = T_PAD - t
    st = jnp.pad(stu_track, (0, pad)).reshape(NW, NCH, CH)
    it = jnp.pad(item_index, (0, pad)).reshape(NW, NCH, CH)
    ci = jnp.pad(conc_index, (0, pad)).reshape(NW, NCH, CH)
    mi = jnp.pad(mean_index, (0, pad),
                 constant_values=N_GROUPS).reshape(NW, NCH, CH)
    zsum = jnp.zeros((G_PAD,), jnp.float32)
    zcnt = jnp.zeros((G_PAD,), jnp.float32)
    sums, counts = _sc_edges(hs, hi, hc, W_pred.reshape(D), zsum, zcnt,
                             st, it, ci, mi)
    pred = _combine(sums, counts, b_pred)
    return pred.reshape(-1)[:N_GROUPS]


# speedup vs baseline: 1.1447x; 1.1447x over previous
import functools

import jax
import jax.numpy as jnp
from jax import lax
from jax.experimental import pallas as pl
from jax.experimental.pallas import tpu as pltpu
from jax.experimental.pallas import tpu_sc as plsc

D = 128
N_GROUPS = 40000
G_PAD = 40960
NW = 32
CH = 128
T_PAD = 163840
EPW = T_PAD // NW
NCH = EPW // CH
NB = D // 32


def _proj_body(stu, conc, item, ws, wi, bs, bi, hs_o, hi_o, hc_o):
    hs_o[...] = jnp.dot(stu[...], ws[...],
                        preferred_element_type=jnp.float32).astype(jnp.bfloat16)
    hi_o[...] = jnp.dot(item[...], wi[...],
                        preferred_element_type=jnp.float32).astype(jnp.bfloat16)
    hc_o[:, :D] = (jnp.dot(conc[...], ws[...],
                           preferred_element_type=jnp.float32)
                   + bs[...]).astype(jnp.bfloat16)
    hc_o[:, D:] = (jnp.dot(conc[...], wi[...],
                           preferred_element_type=jnp.float32)
                   + bi[...]).astype(jnp.bfloat16)


def _project(stu_x, conc_x, item_x, ws, wi, bs, bi):
    n_stu, n_conc, n_item = stu_x.shape[0], conc_x.shape[0], item_x.shape[0]
    return pl.pallas_call(
        _proj_body,
        out_shape=[
            jax.ShapeDtypeStruct((n_stu, D), jnp.bfloat16),
            jax.ShapeDtypeStruct((n_item, D), jnp.bfloat16),
            jax.ShapeDtypeStruct((n_conc, 2 * D), jnp.bfloat16),
        ],
    )(stu_x, conc_x, item_x, ws, wi, bs.reshape(1, D), bi.reshape(1, D))


_MESH = plsc.VectorSubcoreMesh(core_axis_name="c", subcore_axis_name="s")

_GDN = lax.GatherDimensionNumbers(offset_dims=(), collapsed_slice_dims=(0,),
                                  start_index_map=(0,))


def _shuffle(x, idx):
    return lax.gather(x, idx[:, None], dimension_numbers=_GDN,
                      slice_sizes=(1,),
                      mode=lax.GatherScatterMode.PROMISE_IN_BOUNDS)


def _lo(v):
    return lax.bitcast_convert_type(lax.shift_left(v, 16), jnp.float32)


def _hi(v):
    return lax.bitcast_convert_type(
        lax.bitwise_and(v, jnp.int32(-65536)), jnp.float32)


def _sigdiff(a, b):
    ea = jnp.exp(a)
    eb = jnp.exp(b)
    return (ea - eb) / ((1.0 + ea) * (1.0 + eb))


@functools.partial(
    pl.kernel,
    out_type=[
        jax.ShapeDtypeStruct((2, G_PAD), jnp.float32),
        jax.ShapeDtypeStruct((2, G_PAD), jnp.float32),
    ],
    mesh=_MESH,
    compiler_params=pltpu.CompilerParams(use_tc_tiling_on_sc=False),
    scratch_types=[
        pltpu.VMEM((NCH, CH), jnp.int32),
        pltpu.VMEM((NCH, CH), jnp.int32),
        pltpu.VMEM((NCH, CH), jnp.int32),
        pltpu.VMEM((NCH, CH), jnp.int32),
        pltpu.VMEM((CH, D // 2), jnp.int32),
        pltpu.VMEM((CH, D // 2), jnp.int32),
        pltpu.VMEM((CH, D // 2), jnp.int32),
        pltpu.VMEM((CH, D // 2), jnp.int32),
        pltpu.VMEM((CH, D), jnp.int32),
        pltpu.VMEM((CH, D), jnp.int32),
        pltpu.VMEM((CH,), jnp.float32),
        pltpu.VMEM((CH,), jnp.float32),
        pltpu.VMEM((CH,), jnp.float32),
        pltpu.VMEM((D,), jnp.float32),
        pltpu.VMEM_SHARED((G_PAD,), jnp.float32),
        pltpu.VMEM_SHARED((G_PAD,), jnp.float32),
        pltpu.SemaphoreType.DMA,
        pltpu.SemaphoreType.DMA,
        pltpu.SemaphoreType.DMA,
        pltpu.SemaphoreType.DMA,
    ],
)
def _sc_edges(hs, hi, hc, wpred_hbm, zsum_hbm, zcnt_hbm,
              stu_idx, item_idx, conc_idx, mean_idx,
              sums_out, counts_out,
              stu_iv, item_iv, conc_iv, mean_iv,
              rs0, rs1, ri0, ri1, rc0, rc1, sv0, sv1,
              ones_v, w_v,
              sh_sums, sh_counts, sem0, sem1, ssem0, ssem1):
    cid = lax.axis_index("c")
    sid = lax.axis_index("s")
    wid = sid * 2 + cid
    rs, ri, rc, sv = (rs0, rs1), (ri0, ri1), (rc0, rc1), (sv0, sv1)
    sems = (sem0, sem1)
    ssems = (ssem0, ssem1)

    @pl.when(sid == 0)
    def _zero():
        pltpu.sync_copy(zsum_hbm, sh_sums)
        pltpu.sync_copy(zcnt_hbm, sh_counts)

    pltpu.sync_copy(stu_idx.at[wid], stu_iv)
    pltpu.sync_copy(item_idx.at[wid], item_iv)
    pltpu.sync_copy(conc_idx.at[wid], conc_iv)
    pltpu.sync_copy(mean_idx.at[wid], mean_iv)
    pltpu.sync_copy(wpred_hbm, w_v)
    one = jnp.full((16,), 1.0, jnp.float32)
    for j in range(CH // 16):
        ones_v[pl.ds(j * 16, 16)] = one
    wvecs = [w_v[pl.ds(j * 16, 16)] for j in range(D // 16)]
    lane = jnp.arange(16, dtype=jnp.int32)
    perms = [lane ^ (1 << p) for p in range(4)]

    plsc.subcore_barrier()

    def start(k, b):
        pltpu.async_copy(hs.at[stu_iv.at[k]], rs[b], sems[b])
        pltpu.async_copy(hi.at[item_iv.at[k]], ri[b], sems[b])
        pltpu.async_copy(hc.at[conc_iv.at[k]], rc[b], sems[b])

    def drain(k, b):
        pltpu.make_async_copy(hs.at[stu_iv.at[k]], rs[b], sems[b]).wait()
        pltpu.make_async_copy(hi.at[item_iv.at[k]], ri[b], sems[b]).wait()
        pltpu.make_async_copy(hc.at[conc_iv.at[k]], rc[b], sems[b]).wait()

    def drain_scatter(k, b):
        pltpu.make_async_copy(
            sv[b], sh_sums.at[mean_iv.at[k]], ssems[b]).wait()
        pltpu.make_async_copy(
            ones_v, sh_counts.at[mean_iv.at[k]], ssems[b]).wait()

    start(0, 0)

    def outer_body(k0, carry):
        for b in range(2):
            k = k0 * 2 + b
            drain(k, b)

            @pl.when(k + 1 < NCH)
            def _prefetch():
                start(k + 1, 1 - b)

            @pl.when(k >= 2)
            def _sdrain():
                drain_scatter(k - 2, b)

            mrs, mri, mrc = rs[b], ri[b], rc[b]
            msv = sv[b]

            def group_body(g, c2):
                def edge_body(e, svec):
                    t2 = g * 16 + e
                    acc = jnp.zeros((16,), jnp.float32)
                    for j in range(NB):
                        vc = mrc[t2, pl.ds(j * 16, 16)]
                        vs = mrs[t2, pl.ds(j * 16, 16)]
                        vc2 = mrc[t2, pl.ds(D // 2 + j * 16, 16)]
                        vi = mri[t2, pl.ds(j * 16, 16)]
                        a0 = _lo(vc) + _lo(vs)
                        a1 = _hi(vc) + _hi(vs)
                        b0 = _lo(vc2) + _lo(vi)
                        b1 = _hi(vc2) + _hi(vi)
                        acc = acc + wvecs[2 * j] * _sigdiff(a0, b0)
                        acc = acc + wvecs[2 * j + 1] * _sigdiff(a1, b1)
                    for p in perms:
                        acc = acc + _shuffle(acc, p)
                    return jnp.where(lane == e, acc, svec)

                svec = plsc.parallel_loop(
                    0, 16, unroll=4,
                    carry=jnp.zeros((16,), jnp.float32))(edge_body)
                msv[pl.ds(g * 16, 16)] = svec
                return c2

            lax.fori_loop(0, CH // 16, group_body, 0)
            pltpu.async_copy(msv, sh_sums.at[mean_iv.at[k]], ssems[b],
                             add=True)
            pltpu.async_copy(ones_v, sh_counts.at[mean_iv.at[k]], ssems[b],
                             add=True)
        return carry

    lax.fori_loop(0, NCH // 2, outer_body, 0)
    drain_scatter(NCH - 2, 0)
    drain_scatter(NCH - 1, 1)

    plsc.subcore_barrier()

    @pl.when(sid == 0)
    def _flush():
        pltpu.sync_copy(sh_sums, sums_out.at[cid])
        pltpu.sync_copy(sh_counts, counts_out.at[cid])


def _combine_body(s_ref, c_ref, b_ref, o_ref):
    tot = s_ref[0] + s_ref[1]
    cnt = jnp.maximum(c_ref[0] + c_ref[1], 1.0)
    o_ref[...] = jax.nn.sigmoid(tot / cnt + b_ref[0, 0])


def _combine(sums, counts, b_pred):
    return pl.pallas_call(
        _combine_body,
        out_shape=jax.ShapeDtypeStruct((G_PAD // D, D), jnp.float32),
        in_specs=[
            pl.BlockSpec(memory_space=pltpu.VMEM),
            pl.BlockSpec(memory_space=pltpu.VMEM),
            pl.BlockSpec(memory_space=pltpu.SMEM),
        ],
    )(sums.reshape(2, G_PAD // D, D), counts.reshape(2, G_PAD // D, D),
      b_pred.reshape(1, 1))


def kernel(stu_x, conc_x, item_x, stu_track, item_index, conc_index,
           mean_index, W_feat_stu, b_feat_stu, W_feat_item, b_feat_item,
           W_pred, b_pred):
    hs, hi, hc = _project(stu_x, conc_x, item_x,
                          W_feat_stu, W_feat_item, b_feat_stu, b_feat_item)
    hs = lax.bitcast_convert_type(
        hs.reshape(hs.shape[0], D // 2, 2), jnp.int32)
    hi = lax.bitcast_convert_type(
        hi.reshape(hi.shape[0], D // 2, 2), jnp.int32)
    hc = lax.bitcast_convert_type(
        hc.reshape(hc.shape[0], D, 2), jnp.int32)
    t = stu_track.shape[0]
    pad = T_PAD - t
    st = jnp.pad(stu_track, (0, pad)).reshape(NW, NCH, CH)
    it = jnp.pad(item_index, (0, pad)).reshape(NW, NCH, CH)
    ci = jnp.pad(conc_index, (0, pad)).reshape(NW, NCH, CH)
    mi = jnp.pad(mean_index, (0, pad),
                 constant_values=N_GROUPS).reshape(NW, NCH, CH)
    wp = W_pred.reshape(NB, 16, 2).transpose(0, 2, 1).reshape(D)
    zsum = jnp.zeros((G_PAD,), jnp.float32)
    zcnt = jnp.zeros((G_PAD,), jnp.float32)
    sums, counts = _sc_edges(hs, hi, hc, wp, zsum, zcnt, st, it, ci, mi)
    pred = _combine(sums, counts, b_pred)
    return pred.reshape(-1)[:N_GROUPS]


# speedup vs baseline: 1.2630x; 1.1034x over previous
import functools

import jax
import jax.numpy as jnp
from jax import lax
from jax.experimental import pallas as pl
from jax.experimental.pallas import tpu as pltpu
from jax.experimental.pallas import tpu_sc as plsc

D = 128
N_GROUPS = 40000
G_PAD = 40960
NW = 32
CH = 64
NBUF = 4
T_PAD = 163840
EPW = T_PAD // NW
NCH = EPW // CH
NB = D // 32


def _proj_body(stu, conc, item, ws, wi, bs, bi, hs_o, hi_o, hc_o):
    hs_o[...] = jnp.dot(stu[...], ws[...],
                        preferred_element_type=jnp.float32).astype(jnp.bfloat16)
    hi_o[...] = jnp.dot(item[...], wi[...],
                        preferred_element_type=jnp.float32).astype(jnp.bfloat16)
    hc_o[:, :D] = (jnp.dot(conc[...], ws[...],
                           preferred_element_type=jnp.float32)
                   + bs[...]).astype(jnp.bfloat16)
    hc_o[:, D:] = (jnp.dot(conc[...], wi[...],
                           preferred_element_type=jnp.float32)
                   + bi[...]).astype(jnp.bfloat16)


def _project(stu_x, conc_x, item_x, ws, wi, bs, bi):
    n_stu, n_conc, n_item = stu_x.shape[0], conc_x.shape[0], item_x.shape[0]
    return pl.pallas_call(
        _proj_body,
        out_shape=[
            jax.ShapeDtypeStruct((n_stu, D), jnp.bfloat16),
            jax.ShapeDtypeStruct((n_item, D), jnp.bfloat16),
            jax.ShapeDtypeStruct((n_conc, 2 * D), jnp.bfloat16),
        ],
    )(stu_x, conc_x, item_x, ws, wi, bs.reshape(1, D), bi.reshape(1, D))


_MESH = plsc.VectorSubcoreMesh(core_axis_name="c", subcore_axis_name="s")

_GDN = lax.GatherDimensionNumbers(offset_dims=(), collapsed_slice_dims=(0,),
                                  start_index_map=(0,))


def _shuffle(x, idx):
    return lax.gather(x, idx[:, None], dimension_numbers=_GDN,
                      slice_sizes=(1,),
                      mode=lax.GatherScatterMode.PROMISE_IN_BOUNDS)


def _lo(v):
    return lax.bitcast_convert_type(lax.shift_left(v, 16), jnp.float32)


def _hi(v):
    return lax.bitcast_convert_type(
        lax.bitwise_and(v, jnp.int32(-65536)), jnp.float32)


def _sigdiff(a, b):
    ea = jnp.exp(a)
    eb = jnp.exp(b)
    return (ea - eb) / ((1.0 + ea) * (1.0 + eb))


@functools.partial(
    pl.kernel,
    out_type=[
        jax.ShapeDtypeStruct((2, G_PAD), jnp.float32),
        jax.ShapeDtypeStruct((2, G_PAD), jnp.float32),
    ],
    mesh=_MESH,
    compiler_params=pltpu.CompilerParams(use_tc_tiling_on_sc=False),
    scratch_types=[
        pltpu.VMEM((NCH, CH), jnp.int32),
        pltpu.VMEM((NCH, CH), jnp.int32),
        pltpu.VMEM((NCH, CH), jnp.int32),
        pltpu.VMEM((NCH, CH), jnp.int32),
        *([pltpu.VMEM((CH, D // 2), jnp.int32)] * NBUF),
        *([pltpu.VMEM((CH, D // 2), jnp.int32)] * NBUF),
        *([pltpu.VMEM((CH, D), jnp.int32)] * NBUF),
        *([pltpu.VMEM((CH,), jnp.float32)] * NBUF),
        pltpu.VMEM((CH,), jnp.float32),
        pltpu.VMEM((D,), jnp.float32),
        pltpu.VMEM_SHARED((G_PAD,), jnp.float32),
        pltpu.VMEM_SHARED((G_PAD,), jnp.float32),
        *([pltpu.SemaphoreType.DMA] * (2 * NBUF)),
    ],
)
def _sc_edges(hs, hi, hc, wpred_hbm, zsum_hbm, zcnt_hbm,
              stu_idx, item_idx, conc_idx, mean_idx,
              sums_out, counts_out,
              stu_iv, item_iv, conc_iv, mean_iv,
              *rest):
    rs = rest[0:NBUF]
    ri = rest[NBUF:2 * NBUF]
    rc = rest[2 * NBUF:3 * NBUF]
    sv = rest[3 * NBUF:4 * NBUF]
    ones_v, w_v, sh_sums, sh_counts = rest[4 * NBUF:4 * NBUF + 4]
    sems = rest[4 * NBUF + 4:4 * NBUF + 4 + NBUF]
    ssems = rest[4 * NBUF + 4 + NBUF:4 * NBUF + 4 + 2 * NBUF]
    cid = lax.axis_index("c")
    sid = lax.axis_index("s")
    wid = sid * 2 + cid

    @pl.when(sid == 0)
    def _zero():
        pltpu.sync_copy(zsum_hbm, sh_sums)
        pltpu.sync_copy(zcnt_hbm, sh_counts)

    pltpu.sync_copy(stu_idx.at[wid], stu_iv)
    pltpu.sync_copy(item_idx.at[wid], item_iv)
    pltpu.sync_copy(conc_idx.at[wid], conc_iv)
    pltpu.sync_copy(mean_idx.at[wid], mean_iv)
    pltpu.sync_copy(wpred_hbm, w_v)
    one = jnp.full((16,), 1.0, jnp.float32)
    for j in range(CH // 16):
        ones_v[pl.ds(j * 16, 16)] = one
    wvecs = [w_v[pl.ds(j * 16, 16)] for j in range(D // 16)]
    lane = jnp.arange(16, dtype=jnp.int32)
    perms = [lane ^ (1 << p) for p in range(4)]

    plsc.subcore_barrier()

    def start(k, b):
        pltpu.async_copy(hs.at[stu_iv.at[k]], rs[b], sems[b])
        pltpu.async_copy(hi.at[item_iv.at[k]], ri[b], sems[b])
        pltpu.async_copy(hc.at[conc_iv.at[k]], rc[b], sems[b])

    def drain(k, b):
        pltpu.make_async_copy(hs.at[stu_iv.at[k]], rs[b], sems[b]).wait()
        pltpu.make_async_copy(hi.at[item_iv.at[k]], ri[b], sems[b]).wait()
        pltpu.make_async_copy(hc.at[conc_iv.at[k]], rc[b], sems[b]).wait()

    def drain_scatter(k, b):
        pltpu.make_async_copy(
            sv[b], sh_sums.at[mean_iv.at[k]], ssems[b]).wait()
        pltpu.make_async_copy(
            ones_v, sh_counts.at[mean_iv.at[k]], ssems[b]).wait()

    for p in range(NBUF - 1):
        start(p, p)

    def outer_body(k0, carry):
        for b in range(NBUF):
            k = k0 * NBUF + b
            drain(k, b)

            @pl.when(k + NBUF - 1 < NCH)
            def _prefetch():
                start(k + NBUF - 1, (b + NBUF - 1) % NBUF)

            @pl.when(k >= NBUF)
            def _sdrain():
                drain_scatter(k - NBUF, b)

            mrs, mri, mrc = rs[b], ri[b], rc[b]
            msv = sv[b]

            def group_body(g, c2):
                def edge_body(e, svec):
                    t2 = g * 16 + e
                    acc = jnp.zeros((16,), jnp.float32)
                    for j in range(NB):
                        vc = mrc[t2, pl.ds(j * 16, 16)]
                        vs = mrs[t2, pl.ds(j * 16, 16)]
                        vc2 = mrc[t2, pl.ds(D // 2 + j * 16, 16)]
                        vi = mri[t2, pl.ds(j * 16, 16)]
                        a0 = _lo(vc) + _lo(vs)
                        a1 = _hi(vc) + _hi(vs)
                        b0 = _lo(vc2) + _lo(vi)
                        b1 = _hi(vc2) + _hi(vi)
                        acc = acc + wvecs[2 * j] * _sigdiff(a0, b0)
                        acc = acc + wvecs[2 * j + 1] * _sigdiff(a1, b1)
                    for p in perms:
                        acc = acc + _shuffle(acc, p)
                    return jnp.where(lane == e, acc, svec)

                svec = plsc.parallel_loop(
                    0, 16, unroll=4,
                    carry=jnp.zeros((16,), jnp.float32))(edge_body)
                msv[pl.ds(g * 16, 16)] = svec
                return c2

            lax.fori_loop(0, CH // 16, group_body, 0)
            pltpu.async_copy(msv, sh_sums.at[mean_iv.at[k]], ssems[b],
                             add=True)
            pltpu.async_copy(ones_v, sh_counts.at[mean_iv.at[k]], ssems[b],
                             add=True)
        return carry

    lax.fori_loop(0, NCH // NBUF, outer_body, 0)
    for p in range(NBUF):
        drain_scatter(NCH - NBUF + p, (NCH - NBUF + p) % NBUF)

    plsc.subcore_barrier()

    @pl.when(sid == 0)
    def _flush():
        pltpu.sync_copy(sh_sums, sums_out.at[cid])
        pltpu.sync_copy(sh_counts, counts_out.at[cid])


def _combine_body(s_ref, c_ref, b_ref, o_ref):
    tot = s_ref[0] + s_ref[1]
    cnt = jnp.maximum(c_ref[0] + c_ref[1], 1.0)
    o_ref[...] = jax.nn.sigmoid(tot / cnt + b_ref[0, 0])


def _combine(sums, counts, b_pred):
    return pl.pallas_call(
        _combine_body,
        out_shape=jax.ShapeDtypeStruct((G_PAD // D, D), jnp.float32),
        in_specs=[
            pl.BlockSpec(memory_space=pltpu.VMEM),
            pl.BlockSpec(memory_space=pltpu.VMEM),
            pl.BlockSpec(memory_space=pltpu.SMEM),
        ],
    )(sums.reshape(2, G_PAD // D, D), counts.reshape(2, G_PAD // D, D),
      b_pred.reshape(1, 1))


def kernel(stu_x, conc_x, item_x, stu_track, item_index, conc_index,
           mean_index, W_feat_stu, b_feat_stu, W_feat_item, b_feat_item,
           W_pred, b_pred):
    hs, hi, hc = _project(stu_x, conc_x, item_x,
                          W_feat_stu, W_feat_item, b_feat_stu, b_feat_item)
    hs = lax.bitcast_convert_type(
        hs.reshape(hs.shape[0], D // 2, 2), jnp.int32)
    hi = lax.bitcast_convert_type(
        hi.reshape(hi.shape[0], D // 2, 2), jnp.int32)
    hc = lax.bitcast_convert_type(
        hc.reshape(hc.shape[0], D, 2), jnp.int32)
    t = stu_track.shape[0]
    pad = T_PAD - t
    st = jnp.pad(stu_track, (0, pad)).reshape(NW, NCH, CH)
    it = jnp.pad(item_index, (0, pad)).reshape(NW, NCH, CH)
    ci = jnp.pad(conc_index, (0, pad)).reshape(NW, NCH, CH)
    mi = jnp.pad(mean_index, (0, pad),
                 constant_values=N_GROUPS).reshape(NW, NCH, CH)
    wp = W_pred.reshape(NB, 16, 2).transpose(0, 2, 1).reshape(D)
    zsum = jnp.zeros((G_PAD,), jnp.float32)
    zcnt = jnp.zeros((G_PAD,), jnp.float32)
    sums, counts = _sc_edges(hs, hi, hc, wp, zsum, zcnt, st, it, ci, mi)
    pred = _combine(sums, counts, b_pred)
    return pred.reshape(-1)[:N_GROUPS]


# speedup vs baseline: 1.2639x; 1.0007x over previous
import functools

import jax
import jax.numpy as jnp
from jax import lax
from jax.experimental import pallas as pl
from jax.experimental.pallas import tpu as pltpu
from jax.experimental.pallas import tpu_sc as plsc

D = 128
N_GROUPS = 40000
G_PAD = 40960
NW = 32
CH = 64
NBUF = 5
T_PAD = 163840
EPW = T_PAD // NW
NCH = EPW // CH
NB = D // 32


def _proj_body(stu, conc, item, ws, wi, bs, bi, hs_o, hi_o, hc_o):
    hs_o[...] = jnp.dot(stu[...], ws[...],
                        preferred_element_type=jnp.float32).astype(jnp.bfloat16)
    hi_o[...] = jnp.dot(item[...], wi[...],
                        preferred_element_type=jnp.float32).astype(jnp.bfloat16)
    hc_o[:, :D] = (jnp.dot(conc[...], ws[...],
                           preferred_element_type=jnp.float32)
                   + bs[...]).astype(jnp.bfloat16)
    hc_o[:, D:] = (jnp.dot(conc[...], wi[...],
                           preferred_element_type=jnp.float32)
                   + bi[...]).astype(jnp.bfloat16)


def _project(stu_x, conc_x, item_x, ws, wi, bs, bi):
    n_stu, n_conc, n_item = stu_x.shape[0], conc_x.shape[0], item_x.shape[0]
    return pl.pallas_call(
        _proj_body,
        out_shape=[
            jax.ShapeDtypeStruct((n_stu, D), jnp.bfloat16),
            jax.ShapeDtypeStruct((n_item, D), jnp.bfloat16),
            jax.ShapeDtypeStruct((n_conc, 2 * D), jnp.bfloat16),
        ],
    )(stu_x, conc_x, item_x, ws, wi, bs.reshape(1, D), bi.reshape(1, D))


_MESH = plsc.VectorSubcoreMesh(core_axis_name="c", subcore_axis_name="s")

_GDN = lax.GatherDimensionNumbers(offset_dims=(), collapsed_slice_dims=(0,),
                                  start_index_map=(0,))


def _shuffle(x, idx):
    return lax.gather(x, idx[:, None], dimension_numbers=_GDN,
                      slice_sizes=(1,),
                      mode=lax.GatherScatterMode.PROMISE_IN_BOUNDS)


def _lo(v):
    return lax.bitcast_convert_type(lax.shift_left(v, 16), jnp.float32)


def _hi(v):
    return lax.bitcast_convert_type(
        lax.bitwise_and(v, jnp.int32(-65536)), jnp.float32)


def _sigdiff(a, b):
    ea = jnp.exp(a)
    eb = jnp.exp(b)
    return (ea - eb) / ((1.0 + ea) * (1.0 + eb))


@functools.partial(
    pl.kernel,
    out_type=[
        jax.ShapeDtypeStruct((2, G_PAD), jnp.float32),
        jax.ShapeDtypeStruct((2, G_PAD), jnp.float32),
    ],
    mesh=_MESH,
    compiler_params=pltpu.CompilerParams(use_tc_tiling_on_sc=False),
    scratch_types=[
        pltpu.VMEM((NCH, CH), jnp.int32),
        pltpu.VMEM((NCH, CH), jnp.int32),
        pltpu.VMEM((NCH, CH), jnp.int32),
        pltpu.VMEM((NCH, CH), jnp.int32),
        *([pltpu.VMEM((CH, D // 2), jnp.int32)] * NBUF),
        *([pltpu.VMEM((CH, D // 2), jnp.int32)] * NBUF),
        *([pltpu.VMEM((CH, D), jnp.int32)] * NBUF),
        *([pltpu.VMEM((CH,), jnp.float32)] * NBUF),
        pltpu.VMEM((CH,), jnp.float32),
        pltpu.VMEM((D,), jnp.float32),
        pltpu.VMEM_SHARED((G_PAD,), jnp.float32),
        pltpu.VMEM_SHARED((G_PAD,), jnp.float32),
        *([pltpu.SemaphoreType.DMA] * (2 * NBUF)),
    ],
)
def _sc_edges(hs, hi, hc, wpred_hbm, zsum_hbm, zcnt_hbm,
              stu_idx, item_idx, conc_idx, mean_idx,
              sums_out, counts_out,
              stu_iv, item_iv, conc_iv, mean_iv,
              *rest):
    rs = rest[0:NBUF]
    ri = rest[NBUF:2 * NBUF]
    rc = rest[2 * NBUF:3 * NBUF]
    sv = rest[3 * NBUF:4 * NBUF]
    ones_v, w_v, sh_sums, sh_counts = rest[4 * NBUF:4 * NBUF + 4]
    sems = rest[4 * NBUF + 4:4 * NBUF + 4 + NBUF]
    ssems = rest[4 * NBUF + 4 + NBUF:4 * NBUF + 4 + 2 * NBUF]
    cid = lax.axis_index("c")
    sid = lax.axis_index("s")
    wid = sid * 2 + cid

    @pl.when(sid == 0)
    def _zero():
        pltpu.sync_copy(zsum_hbm, sh_sums)
        pltpu.sync_copy(zcnt_hbm, sh_counts)

    pltpu.sync_copy(stu_idx.at[wid], stu_iv)
    pltpu.sync_copy(item_idx.at[wid], item_iv)
    pltpu.sync_copy(conc_idx.at[wid], conc_iv)
    pltpu.sync_copy(mean_idx.at[wid], mean_iv)
    pltpu.sync_copy(wpred_hbm, w_v)
    one = jnp.full((16,), 1.0, jnp.float32)
    for j in range(CH // 16):
        ones_v[pl.ds(j * 16, 16)] = one
    wvecs = [w_v[pl.ds(j * 16, 16)] for j in range(D // 16)]
    lane = jnp.arange(16, dtype=jnp.int32)
    perms = [lane ^ (1 << p) for p in range(4)]

    plsc.subcore_barrier()

    def start(k, b):
        pltpu.async_copy(hs.at[stu_iv.at[k]], rs[b], sems[b])
        pltpu.async_copy(hi.at[item_iv.at[k]], ri[b], sems[b])
        pltpu.async_copy(hc.at[conc_iv.at[k]], rc[b], sems[b])

    def drain(k, b):
        pltpu.make_async_copy(hs.at[stu_iv.at[k]], rs[b], sems[b]).wait()
        pltpu.make_async_copy(hi.at[item_iv.at[k]], ri[b], sems[b]).wait()
        pltpu.make_async_copy(hc.at[conc_iv.at[k]], rc[b], sems[b]).wait()

    def drain_scatter(k, b):
        pltpu.make_async_copy(
            sv[b], sh_sums.at[mean_iv.at[k]], ssems[b]).wait()
        pltpu.make_async_copy(
            ones_v, sh_counts.at[mean_iv.at[k]], ssems[b]).wait()

    for p in range(NBUF - 1):
        start(p, p)

    def outer_body(k0, carry):
        for b in range(NBUF):
            k = k0 * NBUF + b
            drain(k, b)

            @pl.when(k + NBUF - 1 < NCH)
            def _prefetch():
                start(k + NBUF - 1, (b + NBUF - 1) % NBUF)

            @pl.when(k >= NBUF)
            def _sdrain():
                drain_scatter(k - NBUF, b)

            mrs, mri, mrc = rs[b], ri[b], rc[b]
            msv = sv[b]

            def group_body(g, c2):
                def edge_body(e, svec):
                    t2 = g * 16 + e
                    acc = jnp.zeros((16,), jnp.float32)
                    for j in range(NB):
                        vc = mrc[t2, pl.ds(j * 16, 16)]
                        vs = mrs[t2, pl.ds(j * 16, 16)]
                        vc2 = mrc[t2, pl.ds(D // 2 + j * 16, 16)]
                        vi = mri[t2, pl.ds(j * 16, 16)]
                        a0 = _lo(vc) + _lo(vs)
                        a1 = _hi(vc) + _hi(vs)
                        b0 = _lo(vc2) + _lo(vi)
                        b1 = _hi(vc2) + _hi(vi)
                        acc = acc + wvecs[2 * j] * _sigdiff(a0, b0)
                        acc = acc + wvecs[2 * j + 1] * _sigdiff(a1, b1)
                    for p in perms:
                        acc = acc + _shuffle(acc, p)
                    return jnp.where(lane == e, acc, svec)

                svec = plsc.parallel_loop(
                    0, 16, unroll=4,
                    carry=jnp.zeros((16,), jnp.float32))(edge_body)
                msv[pl.ds(g * 16, 16)] = svec
                return c2

            lax.fori_loop(0, CH // 16, group_body, 0)
            pltpu.async_copy(msv, sh_sums.at[mean_iv.at[k]], ssems[b],
                             add=True)
            pltpu.async_copy(ones_v, sh_counts.at[mean_iv.at[k]], ssems[b],
                             add=True)
        return carry

    lax.fori_loop(0, NCH // NBUF, outer_body, 0)
    for p in range(NBUF):
        drain_scatter(NCH - NBUF + p, (NCH - NBUF + p) % NBUF)

    plsc.subcore_barrier()

    @pl.when(sid == 0)
    def _flush():
        pltpu.sync_copy(sh_sums, sums_out.at[cid])
        pltpu.sync_copy(sh_counts, counts_out.at[cid])


def _combine_body(s_ref, c_ref, b_ref, o_ref):
    tot = s_ref[0] + s_ref[1]
    cnt = jnp.maximum(c_ref[0] + c_ref[1], 1.0)
    o_ref[...] = jax.nn.sigmoid(tot / cnt + b_ref[0, 0])


def _combine(sums, counts, b_pred):
    return pl.pallas_call(
        _combine_body,
        out_shape=jax.ShapeDtypeStruct((G_PAD // D, D), jnp.float32),
        in_specs=[
            pl.BlockSpec(memory_space=pltpu.VMEM),
            pl.BlockSpec(memory_space=pltpu.VMEM),
            pl.BlockSpec(memory_space=pltpu.SMEM),
        ],
    )(sums.reshape(2, G_PAD // D, D), counts.reshape(2, G_PAD // D, D),
      b_pred.reshape(1, 1))


def kernel(stu_x, conc_x, item_x, stu_track, item_index, conc_index,
           mean_index, W_feat_stu, b_feat_stu, W_feat_item, b_feat_item,
           W_pred, b_pred):
    hs, hi, hc = _project(stu_x, conc_x, item_x,
                          W_feat_stu, W_feat_item, b_feat_stu, b_feat_item)
    hs = lax.bitcast_convert_type(
        hs.reshape(hs.shape[0], D // 2, 2), jnp.int32)
    hi = lax.bitcast_convert_type(
        hi.reshape(hi.shape[0], D // 2, 2), jnp.int32)
    hc = lax.bitcast_convert_type(
        hc.reshape(hc.shape[0], D, 2), jnp.int32)
    t = stu_track.shape[0]
    pad = T_PAD - t
    st = jnp.pad(stu_track, (0, pad)).reshape(NW, NCH, CH)
    it = jnp.pad(item_index, (0, pad)).reshape(NW, NCH, CH)
    ci = jnp.pad(conc_index, (0, pad)).reshape(NW, NCH, CH)
    mi = jnp.pad(mean_index, (0, pad),
                 constant_values=N_GROUPS).reshape(NW, NCH, CH)
    wp = W_pred.reshape(NB, 16, 2).transpose(0, 2, 1).reshape(D)
    zsum = jnp.zeros((G_PAD,), jnp.float32)
    zcnt = jnp.zeros((G_PAD,), jnp.float32)
    sums, counts = _sc_edges(hs, hi, hc, wp, zsum, zcnt, st, it, ci, mi)
    pred = _combine(sums, counts, b_pred)
    return pred.reshape(-1)[:N_GROUPS]
